# baseline (speedup 1.0000x reference)
import functools
import jax
import jax.numpy as jnp
from jax import lax
from jax.experimental import pallas as pl
from jax.experimental.pallas import tpu as pltpu
from jax.experimental.pallas import tpu_sc as plsc

N = 10000
E = 320000
F = 128

NC = 2
NS = 16
NW = NC * NS
CH = 80
NBUF = 4
NCHUNK = 128
EPW = NCHUNK * CH
EPAD = NW * EPW
TRASH = N
NROWS = 10240
RPT = NROWS // NS


def _sc_agg_body(hp_hbm, src_hbm, dst_hbm, zero_hbm, out_hbm,
                 sidx, didx, rows, acc,
                 gsem0, gsem1, gsem2, gsem3, isem0, isem1, isem2, isem3):
    c = lax.axis_index("c")
    s = lax.axis_index("s")
    w = c * NS + s
    pltpu.sync_copy(zero_hbm, acc.at[pl.ds(s * RPT, RPT)])
    plsc.subcore_barrier()

    gsems = (gsem0, gsem1, gsem2, gsem3)
    isems = (isem0, isem1, isem2, isem3)

    def idx_issue(jj, b):
        pltpu.async_copy(src_hbm.at[w, jj], sidx.at[b], isems[b])
        pltpu.async_copy(dst_hbm.at[w, jj], didx.at[b], isems[b])

    def idx_wait(jj, b):
        pltpu.make_async_copy(src_hbm.at[w, jj], sidx.at[b], isems[b]).wait()
        pltpu.make_async_copy(dst_hbm.at[w, jj], didx.at[b], isems[b]).wait()

    for b in range(NBUF):
        idx_issue(b, b)
    for b in range(NBUF - 1):
        idx_wait(b, b)
        pltpu.async_copy(hp_hbm.at[sidx.at[b]], rows.at[b], gsems[b])

    def round_fn(i, carry):
        for b in range(NBUF):
            j = i * NBUF + b
            b3 = (b + 3) % NBUF

            @pl.when(j + 3 < NCHUNK)
            def _():
                idx_wait(j + 3, b3)
                pltpu.async_copy(
                    hp_hbm.at[sidx.at[b3]], rows.at[b3], gsems[b3])

            pltpu.make_async_copy(
                hp_hbm.at[sidx.at[b]], rows.at[b], gsems[b]).wait()
            pltpu.sync_copy(rows.at[b], acc.at[didx.at[b]], add=True)

            @pl.when(j + NBUF < NCHUNK)
            def _():
                idx_issue(j + NBUF, b)
        return carry

    lax.fori_loop(0, NCHUNK // NBUF, round_fn, 0)
    plsc.subcore_barrier()
    pltpu.sync_copy(acc.at[pl.ds(s * RPT, RPT)],
                    out_hbm.at[c, pl.ds(s * RPT, RPT)])


@functools.lru_cache(maxsize=None)
def _sc_agg_kernel():
    mesh = plsc.VectorSubcoreMesh(
        core_axis_name="c", subcore_axis_name="s",
        num_cores=NC, num_subcores=NS)
    return pl.kernel(
        _sc_agg_body,
        out_type=jax.ShapeDtypeStruct((NC, NROWS, F), jnp.float32),
        mesh=mesh,
        scratch_types=[
            pltpu.VMEM((NBUF, CH), jnp.int32),
            pltpu.VMEM((NBUF, CH), jnp.int32),
            pltpu.VMEM((NBUF, CH, F), jnp.float32),
            pltpu.VMEM_SHARED((NROWS, F), jnp.float32),
            pltpu.SemaphoreType.DMA,
            pltpu.SemaphoreType.DMA,
            pltpu.SemaphoreType.DMA,
            pltpu.SemaphoreType.DMA,
            pltpu.SemaphoreType.DMA,
            pltpu.SemaphoreType.DMA,
            pltpu.SemaphoreType.DMA,
            pltpu.SemaphoreType.DMA,
        ],
    )


def _sc_deg_body(dst_hbm, zero_hbm, ones_hbm, out_hbm,
                 didx, ones_v, acc, sem):
    c = lax.axis_index("c")
    s = lax.axis_index("s")
    w = c * NS + s
    pltpu.sync_copy(ones_hbm, ones_v)
    pltpu.sync_copy(zero_hbm, acc.at[pl.ds(s * RPT, RPT)])
    pltpu.sync_copy(dst_hbm.at[w], didx)
    plsc.subcore_barrier()

    def chunk(j, carry):
        pltpu.sync_copy(ones_v, acc.at[didx.at[j]], add=True)
        return carry

    lax.fori_loop(0, NCHUNK, chunk, 0)
    plsc.subcore_barrier()
    pltpu.sync_copy(acc.at[pl.ds(s * RPT, RPT)],
                    out_hbm.at[c, pl.ds(s * RPT, RPT)])


@functools.lru_cache(maxsize=None)
def _sc_deg_kernel():
    mesh = plsc.VectorSubcoreMesh(
        core_axis_name="c", subcore_axis_name="s",
        num_cores=NC, num_subcores=NS)
    return pl.kernel(
        _sc_deg_body,
        out_type=jax.ShapeDtypeStruct((NC, NROWS, F), jnp.float32),
        mesh=mesh,
        scratch_types=[
            pltpu.VMEM((NCHUNK, CH), jnp.int32),
            pltpu.VMEM((CH, F), jnp.float32),
            pltpu.VMEM_SHARED((NROWS, F), jnp.float32),
            pltpu.SemaphoreType.DMA,
        ],
    )


BR = 512
GRID = NROWS // BR


def _dis_block(degp):
    deg = degp[0, :, 0:1] + degp[1, :, 0:1] + 1.0
    return lax.rsqrt(deg)


def _tc_first_body(x_ref, w_ref, degp_ref, out_ref):
    dis = _dis_block(degp_ref[...])
    h = jnp.dot(x_ref[...], w_ref[...], preferred_element_type=jnp.float32)
    out_ref[...] = h * dis


def _tc_mid_body(accp_ref, hp_ref, degp_ref, b_ref, w_ref, out_ref):
    dis = _dis_block(degp_ref[...])
    accp = accp_ref[...]
    a = jnp.tanh((accp[0] + accp[1] + hp_ref[...]) * dis + b_ref[...])
    out_ref[...] = jnp.dot(a, w_ref[...],
                           preferred_element_type=jnp.float32) * dis


def _tc_last_body(accp_ref, hp_ref, degp_ref, b_ref, out_ref):
    dis = _dis_block(degp_ref[...])
    accp = accp_ref[...]
    out_ref[...] = (accp[0] + accp[1] + hp_ref[...]) * dis + b_ref[...]


_row_spec = pl.BlockSpec((BR, F), lambda i: (i, 0))
_acc_spec = pl.BlockSpec((NC, BR, F), lambda i: (0, i, 0))
_deg_spec = pl.BlockSpec((NC, BR, F), lambda i: (0, i, 0))
_w_spec = pl.BlockSpec((F, F), lambda i: (0, 0))
_b_spec = pl.BlockSpec((1, F), lambda i: (0, 0))
_out_sd = jax.ShapeDtypeStruct((NROWS, F), jnp.float32)

_tc_first = pl.pallas_call(
    _tc_first_body, grid=(GRID,),
    in_specs=[_row_spec, _w_spec, _deg_spec],
    out_specs=_row_spec, out_shape=_out_sd)

_tc_mid = pl.pallas_call(
    _tc_mid_body, grid=(GRID,),
    in_specs=[_acc_spec, _row_spec, _deg_spec, _b_spec, _w_spec],
    out_specs=_row_spec, out_shape=_out_sd)

_tc_last = pl.pallas_call(
    _tc_last_body, grid=(GRID,),
    in_specs=[_acc_spec, _row_spec, _deg_spec, _b_spec],
    out_specs=_row_spec, out_shape=_out_sd)


@jax.jit
def kernel(t, x, edge_index, W1, b1, W2, b2, W3, b3):
    del t
    src = edge_index[0]
    dst = edge_index[1]
    pad = EPAD - E
    srcp = jnp.concatenate(
        [src, jnp.zeros((pad,), jnp.int32)]).reshape(NW, NCHUNK, CH)
    dstp = jnp.concatenate(
        [dst, jnp.full((pad,), TRASH, jnp.int32)]).reshape(NW, NCHUNK, CH)

    xp = jnp.pad(x, ((0, NROWS - N), (0, 0)))
    zero_f = jnp.zeros((RPT, F), jnp.float32)
    ones_d = jnp.ones((CH, F), jnp.float32)
    b1r = b1.reshape(1, F)
    b2r = b2.reshape(1, F)
    b3r = b3.reshape(1, F)

    sc_deg = _sc_deg_kernel()
    sc_agg = _sc_agg_kernel()
    degp = sc_deg(dstp, zero_f, ones_d)

    hp1 = _tc_first(xp, W1, degp)
    acc1 = sc_agg(hp1, srcp, dstp, zero_f)
    hp2 = _tc_mid(acc1, hp1, degp, b1r, W2)
    acc2 = sc_agg(hp2, srcp, dstp, zero_f)
    hp3 = _tc_mid(acc2, hp2, degp, b2r, W3)
    acc3 = sc_agg(hp3, srcp, dstp, zero_f)
    out = _tc_last(acc3, hp3, degp, b3r)
    return out[:N]


# speedup vs baseline: 1.8948x; 1.8948x over previous
import functools
import jax
import jax.numpy as jnp
from jax import lax
from jax.experimental import pallas as pl
from jax.experimental.pallas import tpu as pltpu
from jax.experimental.pallas import tpu_sc as plsc

N = 10000
E = 320000
F = 128

NC = 2
NS = 16
NW = NC * NS
CH = 80
NBUF = 3
NCHUNK = 126
EPW = NCHUNK * CH
EPAD = NW * EPW
TRASH = N
NROWS = 10240
RPT = NROWS // NS


def _sc_agg_body(hp_hbm, src_hbm, dst_hbm, zero_hbm, out_hbm,
                 sidx, didx, rows, acc,
                 gsem0, gsem1, gsem2, isem0, isem1, isem2,
                 ssem0, ssem1, ssem2):
    c = lax.axis_index("c")
    s = lax.axis_index("s")
    w = c * NS + s
    pltpu.sync_copy(zero_hbm, acc.at[pl.ds(s * RPT, RPT)])
    plsc.subcore_barrier()

    gsems = (gsem0, gsem1, gsem2)
    isems = (isem0, isem1, isem2)
    ssems = (ssem0, ssem1, ssem2)

    pltpu.sync_copy(dst_hbm.at[w], didx)

    for b in range(NBUF):
        pltpu.async_copy(src_hbm.at[w, b], sidx.at[b], isems[b])
    for b in range(NBUF - 1):
        pltpu.make_async_copy(src_hbm.at[w, b], sidx.at[b], isems[b]).wait()
        pltpu.async_copy(hp_hbm.at[sidx.at[b]], rows.at[b], gsems[b])

    def round_fn(i, carry):
        for b in range(NBUF):
            j = i * NBUF + b
            bg = (b + NBUF - 1) % NBUF

            @pl.when(j + NBUF - 1 < NCHUNK)
            def _():
                pltpu.make_async_copy(
                    src_hbm.at[w, j + NBUF - 1],
                    sidx.at[bg], isems[bg]).wait()

                @pl.when(j >= 1)
                def _():
                    pltpu.make_async_copy(
                        rows.at[bg], acc.at[didx.at[bg]], ssems[bg]).wait()

                pltpu.async_copy(
                    hp_hbm.at[sidx.at[bg]], rows.at[bg], gsems[bg])

            pltpu.make_async_copy(
                hp_hbm.at[sidx.at[b]], rows.at[b], gsems[b]).wait()
            pltpu.async_copy(rows.at[b], acc.at[didx.at[j]], ssems[b],
                             add=True)

            @pl.when(j + NBUF < NCHUNK)
            def _():
                pltpu.async_copy(
                    src_hbm.at[w, j + NBUF], sidx.at[b], isems[b])
        return carry

    lax.fori_loop(0, NCHUNK // NBUF, round_fn, 0)
    for b in range(NBUF):
        pltpu.make_async_copy(
            rows.at[b], acc.at[didx.at[b]], ssems[b]).wait()
    plsc.subcore_barrier()
    pltpu.sync_copy(acc.at[pl.ds(s * RPT, RPT)],
                    out_hbm.at[c, pl.ds(s * RPT, RPT)])


@functools.lru_cache(maxsize=None)
def _sc_agg_kernel():
    mesh = plsc.VectorSubcoreMesh(
        core_axis_name="c", subcore_axis_name="s",
        num_cores=NC, num_subcores=NS)
    return pl.kernel(
        _sc_agg_body,
        out_type=jax.ShapeDtypeStruct((NC, NROWS, F), jnp.float32),
        mesh=mesh,
        scratch_types=[
            pltpu.VMEM((NBUF, CH), jnp.int32),
            pltpu.VMEM((NCHUNK, CH), jnp.int32),
            pltpu.VMEM((NBUF, CH, F), jnp.float32),
            pltpu.VMEM_SHARED((NROWS, F), jnp.float32),
            pltpu.SemaphoreType.DMA,
            pltpu.SemaphoreType.DMA,
            pltpu.SemaphoreType.DMA,
            pltpu.SemaphoreType.DMA,
            pltpu.SemaphoreType.DMA,
            pltpu.SemaphoreType.DMA,
            pltpu.SemaphoreType.DMA,
            pltpu.SemaphoreType.DMA,
            pltpu.SemaphoreType.DMA,
        ],
    )


def _sc_deg_body(dst_hbm, zero_hbm, ones_hbm, out_hbm,
                 didx, ones_v, acc, sem):
    c = lax.axis_index("c")
    s = lax.axis_index("s")
    w = c * NS + s
    pltpu.sync_copy(ones_hbm, ones_v)
    pltpu.sync_copy(zero_hbm, acc.at[pl.ds(s * RPT, RPT)])
    pltpu.sync_copy(dst_hbm.at[w], didx)
    plsc.subcore_barrier()

    def chunk(j, carry):
        pltpu.sync_copy(ones_v, acc.at[didx.at[j]], add=True)
        return carry

    lax.fori_loop(0, NCHUNK, chunk, 0)
    plsc.subcore_barrier()
    pltpu.sync_copy(acc.at[pl.ds(s * RPT, RPT)],
                    out_hbm.at[c, pl.ds(s * RPT, RPT)])


@functools.lru_cache(maxsize=None)
def _sc_deg_kernel():
    mesh = plsc.VectorSubcoreMesh(
        core_axis_name="c", subcore_axis_name="s",
        num_cores=NC, num_subcores=NS)
    return pl.kernel(
        _sc_deg_body,
        out_type=jax.ShapeDtypeStruct((NC, NROWS, F), jnp.float32),
        mesh=mesh,
        scratch_types=[
            pltpu.VMEM((NCHUNK, CH), jnp.int32),
            pltpu.VMEM((CH, F), jnp.float32),
            pltpu.VMEM_SHARED((NROWS, F), jnp.float32),
            pltpu.SemaphoreType.DMA,
        ],
    )


BR = 512
GRID = NROWS // BR


def _dis_block(degp):
    deg = degp[0, :, 0:1] + degp[1, :, 0:1] + 1.0
    return lax.rsqrt(deg)


def _tc_first_body(x_ref, w_ref, degp_ref, out_ref):
    dis = _dis_block(degp_ref[...])
    h = jnp.dot(x_ref[...], w_ref[...], preferred_element_type=jnp.float32)
    out_ref[...] = h * dis


def _tc_mid_body(accp_ref, hp_ref, degp_ref, b_ref, w_ref, out_ref):
    dis = _dis_block(degp_ref[...])
    accp = accp_ref[...]
    a = jnp.tanh((accp[0] + accp[1] + hp_ref[...]) * dis + b_ref[...])
    out_ref[...] = jnp.dot(a, w_ref[...],
                           preferred_element_type=jnp.float32) * dis


def _tc_last_body(accp_ref, hp_ref, degp_ref, b_ref, out_ref):
    dis = _dis_block(degp_ref[...])
    accp = accp_ref[...]
    out_ref[...] = (accp[0] + accp[1] + hp_ref[...]) * dis + b_ref[...]


_row_spec = pl.BlockSpec((BR, F), lambda i: (i, 0))
_acc_spec = pl.BlockSpec((NC, BR, F), lambda i: (0, i, 0))
_deg_spec = pl.BlockSpec((NC, BR, F), lambda i: (0, i, 0))
_w_spec = pl.BlockSpec((F, F), lambda i: (0, 0))
_b_spec = pl.BlockSpec((1, F), lambda i: (0, 0))
_out_sd = jax.ShapeDtypeStruct((NROWS, F), jnp.float32)

_tc_first = pl.pallas_call(
    _tc_first_body, grid=(GRID,),
    in_specs=[_row_spec, _w_spec, _deg_spec],
    out_specs=_row_spec, out_shape=_out_sd)

_tc_mid = pl.pallas_call(
    _tc_mid_body, grid=(GRID,),
    in_specs=[_acc_spec, _row_spec, _deg_spec, _b_spec, _w_spec],
    out_specs=_row_spec, out_shape=_out_sd)

_tc_last = pl.pallas_call(
    _tc_last_body, grid=(GRID,),
    in_specs=[_acc_spec, _row_spec, _deg_spec, _b_spec],
    out_specs=_row_spec, out_shape=_out_sd)


@jax.jit
def kernel(t, x, edge_index, W1, b1, W2, b2, W3, b3):
    del t
    src = edge_index[0]
    dst = edge_index[1]
    pad = EPAD - E
    srcp = jnp.concatenate(
        [src, jnp.zeros((pad,), jnp.int32)]).reshape(NW, NCHUNK, CH)
    dstp = jnp.concatenate(
        [dst, jnp.full((pad,), TRASH, jnp.int32)]).reshape(NW, NCHUNK, CH)

    xp = jnp.pad(x, ((0, NROWS - N), (0, 0)))
    zero_f = jnp.zeros((RPT, F), jnp.float32)
    ones_d = jnp.ones((CH, F), jnp.float32)
    b1r = b1.reshape(1, F)
    b2r = b2.reshape(1, F)
    b3r = b3.reshape(1, F)

    sc_deg = _sc_deg_kernel()
    sc_agg = _sc_agg_kernel()
    degp = sc_deg(dstp, zero_f, ones_d)

    hp1 = _tc_first(xp, W1, degp)
    acc1 = sc_agg(hp1, srcp, dstp, zero_f)
    hp2 = _tc_mid(acc1, hp1, degp, b1r, W2)
    acc2 = sc_agg(hp2, srcp, dstp, zero_f)
    hp3 = _tc_mid(acc2, hp2, degp, b2r, W3)
    acc3 = sc_agg(hp3, srcp, dstp, zero_f)
    out = _tc_last(acc3, hp3, degp, b3r)
    return out[:N]


# speedup vs baseline: 1.8990x; 1.0022x over previous
import functools
import jax
import jax.numpy as jnp
from jax import lax
from jax.experimental import pallas as pl
from jax.experimental.pallas import tpu as pltpu
from jax.experimental.pallas import tpu_sc as plsc

N = 10000
E = 320000
F = 128

NC = 2
NS = 16
NW = NC * NS
CH = 80
NBUF = 3
NCHUNK = 126
EPW = NCHUNK * CH
EPAD = NW * EPW
TRASH = N
NROWS = 10240
RPT = NROWS // NS


def _sc_agg_body(hp_hbm, src_hbm, dst_hbm, zero_hbm, out_hbm,
                 sidx, didx, rows, acc,
                 gsem0, gsem1, gsem2, isem0, isem1, isem2,
                 ssem0, ssem1, ssem2):
    c = lax.axis_index("c")
    s = lax.axis_index("s")
    w = c * NS + s
    pltpu.sync_copy(zero_hbm, acc.at[pl.ds(s * RPT, RPT)])
    plsc.subcore_barrier()

    gsems = (gsem0, gsem1, gsem2)
    isems = (isem0, isem1, isem2)
    ssems = (ssem0, ssem1, ssem2)
    HC = CH // 2

    def gather_issue(b):
        pltpu.async_copy(hp_hbm.at[sidx.at[b]], rows.at[b], gsems[b])

    def gather_wait(b):
        pltpu.make_async_copy(
            hp_hbm.at[sidx.at[b]], rows.at[b], gsems[b]).wait()

    pltpu.sync_copy(dst_hbm.at[w], didx)

    for b in range(NBUF):
        pltpu.async_copy(src_hbm.at[w, b], sidx.at[b], isems[b])
    for b in range(NBUF - 1):
        pltpu.make_async_copy(src_hbm.at[w, b], sidx.at[b], isems[b]).wait()
        gather_issue(b)

    def round_fn(i, carry):
        for b in range(NBUF):
            j = i * NBUF + b
            bg = (b + NBUF - 1) % NBUF

            @pl.when(j + NBUF - 1 < NCHUNK)
            def _():
                pltpu.make_async_copy(
                    src_hbm.at[w, j + NBUF - 1],
                    sidx.at[bg], isems[bg]).wait()

                @pl.when(j >= 1)
                def _():
                    pltpu.make_async_copy(
                        rows.at[bg], acc.at[didx.at[bg]], ssems[bg]).wait()

                gather_issue(bg)

            gather_wait(b)
            pltpu.async_copy(rows.at[b], acc.at[didx.at[j]], ssems[b],
                             add=True)

            @pl.when(j + NBUF < NCHUNK)
            def _():
                pltpu.async_copy(
                    src_hbm.at[w, j + NBUF], sidx.at[b], isems[b])
        return carry

    lax.fori_loop(0, NCHUNK // NBUF, round_fn, 0)
    for b in range(NBUF):
        pltpu.make_async_copy(
            rows.at[b], acc.at[didx.at[b]], ssems[b]).wait()
    plsc.subcore_barrier()
    pltpu.sync_copy(acc.at[pl.ds(s * RPT, RPT)],
                    out_hbm.at[c, pl.ds(s * RPT, RPT)])


@functools.lru_cache(maxsize=None)
def _sc_agg_kernel():
    mesh = plsc.VectorSubcoreMesh(
        core_axis_name="c", subcore_axis_name="s",
        num_cores=NC, num_subcores=NS)
    return pl.kernel(
        _sc_agg_body,
        out_type=jax.ShapeDtypeStruct((NC, NROWS, F), jnp.float32),
        mesh=mesh,
        scratch_types=[
            pltpu.VMEM((NBUF, CH), jnp.int32),
            pltpu.VMEM((NCHUNK, CH), jnp.int32),
            pltpu.VMEM((NBUF, CH, F), jnp.float32),
            pltpu.VMEM_SHARED((NROWS, F), jnp.float32),
            pltpu.SemaphoreType.DMA,
            pltpu.SemaphoreType.DMA,
            pltpu.SemaphoreType.DMA,
            pltpu.SemaphoreType.DMA,
            pltpu.SemaphoreType.DMA,
            pltpu.SemaphoreType.DMA,
            pltpu.SemaphoreType.DMA,
            pltpu.SemaphoreType.DMA,
            pltpu.SemaphoreType.DMA,
        ],
    )


def _sc_deg_body(dst_hbm, zero_hbm, ones_hbm, out_hbm,
                 didx, ones_v, acc, dsem0, dsem1, dsem2):
    c = lax.axis_index("c")
    s = lax.axis_index("s")
    w = c * NS + s
    pltpu.sync_copy(ones_hbm, ones_v)
    pltpu.sync_copy(zero_hbm, acc.at[pl.ds(s * RPT, RPT)])
    pltpu.sync_copy(dst_hbm.at[w], didx)
    plsc.subcore_barrier()

    dsems = (dsem0, dsem1, dsem2)

    def chunk(j, carry):
        for p in range(NBUF):
            @pl.when(j + p >= NBUF)
            def _():
                pltpu.make_async_copy(
                    ones_v, acc.at[didx.at[p]], dsems[p]).wait()
            pltpu.async_copy(
                ones_v, acc.at[didx.at[j + p]], dsems[p], add=True)
        return carry

    lax.fori_loop(0, NCHUNK // NBUF, lambda i, car: chunk(i * NBUF, car), 0)
    for p in range(NBUF):
        pltpu.make_async_copy(ones_v, acc.at[didx.at[p]], dsems[p]).wait()
    plsc.subcore_barrier()
    pltpu.sync_copy(acc.at[pl.ds(s * RPT, RPT)],
                    out_hbm.at[c, pl.ds(s * RPT, RPT)])


@functools.lru_cache(maxsize=None)
def _sc_deg_kernel():
    mesh = plsc.VectorSubcoreMesh(
        core_axis_name="c", subcore_axis_name="s",
        num_cores=NC, num_subcores=NS)
    return pl.kernel(
        _sc_deg_body,
        out_type=jax.ShapeDtypeStruct((NC, NROWS, F), jnp.float32),
        mesh=mesh,
        scratch_types=[
            pltpu.VMEM((NCHUNK, CH), jnp.int32),
            pltpu.VMEM((CH, F), jnp.float32),
            pltpu.VMEM_SHARED((NROWS, F), jnp.float32),
            pltpu.SemaphoreType.DMA,
            pltpu.SemaphoreType.DMA,
            pltpu.SemaphoreType.DMA,
        ],
    )


BR = 512
GRID = NROWS // BR


def _dis_block(degp):
    deg = degp[0, :, 0:1] + degp[1, :, 0:1] + 1.0
    return lax.rsqrt(deg)


def _tc_first_body(x_ref, w_ref, degp_ref, out_ref):
    dis = _dis_block(degp_ref[...])
    h = jnp.dot(x_ref[...], w_ref[...], preferred_element_type=jnp.float32)
    out_ref[...] = h * dis


def _tc_mid_body(accp_ref, hp_ref, degp_ref, b_ref, w_ref, out_ref):
    dis = _dis_block(degp_ref[...])
    accp = accp_ref[...]
    a = jnp.tanh((accp[0] + accp[1] + hp_ref[...]) * dis + b_ref[...])
    out_ref[...] = jnp.dot(a, w_ref[...],
                           preferred_element_type=jnp.float32) * dis


def _tc_last_body(accp_ref, hp_ref, degp_ref, b_ref, out_ref):
    dis = _dis_block(degp_ref[...])
    accp = accp_ref[...]
    out_ref[...] = (accp[0] + accp[1] + hp_ref[...]) * dis + b_ref[...]


_row_spec = pl.BlockSpec((BR, F), lambda i: (i, 0))
_acc_spec = pl.BlockSpec((NC, BR, F), lambda i: (0, i, 0))
_deg_spec = pl.BlockSpec((NC, BR, F), lambda i: (0, i, 0))
_w_spec = pl.BlockSpec((F, F), lambda i: (0, 0))
_b_spec = pl.BlockSpec((1, F), lambda i: (0, 0))
_out_sd = jax.ShapeDtypeStruct((NROWS, F), jnp.float32)

_tc_first = pl.pallas_call(
    _tc_first_body, grid=(GRID,),
    in_specs=[_row_spec, _w_spec, _deg_spec],
    out_specs=_row_spec, out_shape=_out_sd)

_tc_mid = pl.pallas_call(
    _tc_mid_body, grid=(GRID,),
    in_specs=[_acc_spec, _row_spec, _deg_spec, _b_spec, _w_spec],
    out_specs=_row_spec, out_shape=_out_sd)

_tc_last = pl.pallas_call(
    _tc_last_body, grid=(GRID,),
    in_specs=[_acc_spec, _row_spec, _deg_spec, _b_spec],
    out_specs=_row_spec, out_shape=_out_sd)


@jax.jit
def kernel(t, x, edge_index, W1, b1, W2, b2, W3, b3):
    del t
    src = edge_index[0]
    dst = edge_index[1]
    pad = EPAD - E
    srcp = jnp.concatenate(
        [src, jnp.zeros((pad,), jnp.int32)]).reshape(NW, NCHUNK, CH)
    dstp = jnp.concatenate(
        [dst, jnp.full((pad,), TRASH, jnp.int32)]).reshape(NW, NCHUNK, CH)

    xp = jnp.pad(x, ((0, NROWS - N), (0, 0)))
    zero_f = jnp.zeros((RPT, F), jnp.float32)
    ones_d = jnp.ones((CH, F), jnp.float32)
    b1r = b1.reshape(1, F)
    b2r = b2.reshape(1, F)
    b3r = b3.reshape(1, F)

    sc_deg = _sc_deg_kernel()
    sc_agg = _sc_agg_kernel()
    degp = sc_deg(dstp, zero_f, ones_d)

    hp1 = _tc_first(xp, W1, degp)
    acc1 = sc_agg(hp1, srcp, dstp, zero_f)
    hp2 = _tc_mid(acc1, hp1, degp, b1r, W2)
    acc2 = sc_agg(hp2, srcp, dstp, zero_f)
    hp3 = _tc_mid(acc2, hp2, degp, b2r, W3)
    acc3 = sc_agg(hp3, srcp, dstp, zero_f)
    out = _tc_last(acc3, hp3, degp, b3r)
    return out[:N]


# speedup vs baseline: 1.9081x; 1.0048x over previous
import functools
import jax
import jax.numpy as jnp
from jax import lax
from jax.experimental import pallas as pl
from jax.experimental.pallas import tpu as pltpu
from jax.experimental.pallas import tpu_sc as plsc

N = 10000
E = 320000
F = 128

NC = 2
NS = 16
NW = NC * NS
CH = 80
NBUF = 3
NCHUNK = 126
EPW = NCHUNK * CH
EPAD = NW * EPW
TRASH = N
NROWS = 10240
RPT = NROWS // NS


def _sc_agg_body(hp_hbm, src_hbm, dst_hbm, zero_hbm, out_hbm,
                 sidx, didx, rows, acc,
                 gsem0, gsem1, gsem2, isem0, isem1, isem2,
                 ssem0, ssem1, ssem2):
    c = lax.axis_index("c")
    s = lax.axis_index("s")
    w = c * NS + s
    pltpu.sync_copy(zero_hbm, acc.at[pl.ds(s * RPT, RPT)])
    plsc.subcore_barrier()

    gsems = (gsem0, gsem1, gsem2)
    isems = (isem0, isem1, isem2)
    ssems = (ssem0, ssem1, ssem2)
    HC = CH // 2

    def gather_issue(b):
        pltpu.async_copy(hp_hbm.at[sidx.at[b]], rows.at[b], gsems[b])

    def gather_wait(b):
        pltpu.make_async_copy(
            hp_hbm.at[sidx.at[b]], rows.at[b], gsems[b]).wait()

    pltpu.sync_copy(dst_hbm.at[w], didx)

    for b in range(NBUF):
        pltpu.async_copy(src_hbm.at[w, b], sidx.at[b], isems[b])
    for b in range(NBUF - 1):
        pltpu.make_async_copy(src_hbm.at[w, b], sidx.at[b], isems[b]).wait()
        gather_issue(b)

    def round_fn(i, carry):
        for b in range(NBUF):
            j = i * NBUF + b
            bg = (b + NBUF - 1) % NBUF

            gather_wait(b)
            pltpu.async_copy(rows.at[b], acc.at[didx.at[j]], ssems[b],
                             add=True)

            @pl.when(j + NBUF - 1 < NCHUNK)
            def _():
                pltpu.make_async_copy(
                    src_hbm.at[w, j + NBUF - 1],
                    sidx.at[bg], isems[bg]).wait()

                @pl.when(j >= 1)
                def _():
                    pltpu.make_async_copy(
                        rows.at[bg], acc.at[didx.at[bg]], ssems[bg]).wait()

                gather_issue(bg)

            @pl.when(j + NBUF < NCHUNK)
            def _():
                pltpu.async_copy(
                    src_hbm.at[w, j + NBUF], sidx.at[b], isems[b])
        return carry

    lax.fori_loop(0, NCHUNK // NBUF, round_fn, 0)
    for b in range(NBUF):
        pltpu.make_async_copy(
            rows.at[b], acc.at[didx.at[b]], ssems[b]).wait()
    plsc.subcore_barrier()
    pltpu.sync_copy(acc.at[pl.ds(s * RPT, RPT)],
                    out_hbm.at[c, pl.ds(s * RPT, RPT)])


@functools.lru_cache(maxsize=None)
def _sc_agg_kernel():
    mesh = plsc.VectorSubcoreMesh(
        core_axis_name="c", subcore_axis_name="s",
        num_cores=NC, num_subcores=NS)
    return pl.kernel(
        _sc_agg_body,
        out_type=jax.ShapeDtypeStruct((NC, NROWS, F), jnp.float32),
        mesh=mesh,
        scratch_types=[
            pltpu.VMEM((NBUF, CH), jnp.int32),
            pltpu.VMEM((NCHUNK, CH), jnp.int32),
            pltpu.VMEM((NBUF, CH, F), jnp.float32),
            pltpu.VMEM_SHARED((NROWS, F), jnp.float32),
            pltpu.SemaphoreType.DMA,
            pltpu.SemaphoreType.DMA,
            pltpu.SemaphoreType.DMA,
            pltpu.SemaphoreType.DMA,
            pltpu.SemaphoreType.DMA,
            pltpu.SemaphoreType.DMA,
            pltpu.SemaphoreType.DMA,
            pltpu.SemaphoreType.DMA,
            pltpu.SemaphoreType.DMA,
        ],
    )


def _sc_deg_body(dst_hbm, zero_hbm, ones_hbm, out_hbm,
                 didx, ones_v, acc, dsem0, dsem1, dsem2):
    c = lax.axis_index("c")
    s = lax.axis_index("s")
    w = c * NS + s
    pltpu.sync_copy(ones_hbm, ones_v)
    pltpu.sync_copy(zero_hbm, acc.at[pl.ds(s * RPT, RPT)])
    pltpu.sync_copy(dst_hbm.at[w], didx)
    plsc.subcore_barrier()

    dsems = (dsem0, dsem1, dsem2)

    def chunk(j, carry):
        for p in range(NBUF):
            @pl.when(j + p >= NBUF)
            def _():
                pltpu.make_async_copy(
                    ones_v, acc.at[didx.at[p]], dsems[p]).wait()
            pltpu.async_copy(
                ones_v, acc.at[didx.at[j + p]], dsems[p], add=True)
        return carry

    lax.fori_loop(0, NCHUNK // NBUF, lambda i, car: chunk(i * NBUF, car), 0)
    for p in range(NBUF):
        pltpu.make_async_copy(ones_v, acc.at[didx.at[p]], dsems[p]).wait()
    plsc.subcore_barrier()
    pltpu.sync_copy(acc.at[pl.ds(s * RPT, RPT)],
                    out_hbm.at[c, pl.ds(s * RPT, RPT)])


@functools.lru_cache(maxsize=None)
def _sc_deg_kernel():
    mesh = plsc.VectorSubcoreMesh(
        core_axis_name="c", subcore_axis_name="s",
        num_cores=NC, num_subcores=NS)
    return pl.kernel(
        _sc_deg_body,
        out_type=jax.ShapeDtypeStruct((NC, NROWS, F), jnp.float32),
        mesh=mesh,
        scratch_types=[
            pltpu.VMEM((NCHUNK, CH), jnp.int32),
            pltpu.VMEM((CH, F), jnp.float32),
            pltpu.VMEM_SHARED((NROWS, F), jnp.float32),
            pltpu.SemaphoreType.DMA,
            pltpu.SemaphoreType.DMA,
            pltpu.SemaphoreType.DMA,
        ],
    )


BR = 512
GRID = NROWS // BR


def _dis_block(degp):
    deg = degp[0, :, 0:1] + degp[1, :, 0:1] + 1.0
    return lax.rsqrt(deg)


def _tc_first_body(x_ref, w_ref, degp_ref, out_ref):
    dis = _dis_block(degp_ref[...])
    h = jnp.dot(x_ref[...], w_ref[...], preferred_element_type=jnp.float32)
    out_ref[...] = h * dis


def _tc_mid_body(accp_ref, hp_ref, degp_ref, b_ref, w_ref, out_ref):
    dis = _dis_block(degp_ref[...])
    accp = accp_ref[...]
    a = jnp.tanh((accp[0] + accp[1] + hp_ref[...]) * dis + b_ref[...])
    out_ref[...] = jnp.dot(a, w_ref[...],
                           preferred_element_type=jnp.float32) * dis


def _tc_last_body(accp_ref, hp_ref, degp_ref, b_ref, out_ref):
    dis = _dis_block(degp_ref[...])
    accp = accp_ref[...]
    out_ref[...] = (accp[0] + accp[1] + hp_ref[...]) * dis + b_ref[...]


_row_spec = pl.BlockSpec((BR, F), lambda i: (i, 0))
_acc_spec = pl.BlockSpec((NC, BR, F), lambda i: (0, i, 0))
_deg_spec = pl.BlockSpec((NC, BR, F), lambda i: (0, i, 0))
_w_spec = pl.BlockSpec((F, F), lambda i: (0, 0))
_b_spec = pl.BlockSpec((1, F), lambda i: (0, 0))
_out_sd = jax.ShapeDtypeStruct((NROWS, F), jnp.float32)

_tc_first = pl.pallas_call(
    _tc_first_body, grid=(GRID,),
    in_specs=[_row_spec, _w_spec, _deg_spec],
    out_specs=_row_spec, out_shape=_out_sd)

_tc_mid = pl.pallas_call(
    _tc_mid_body, grid=(GRID,),
    in_specs=[_acc_spec, _row_spec, _deg_spec, _b_spec, _w_spec],
    out_specs=_row_spec, out_shape=_out_sd)

_tc_last = pl.pallas_call(
    _tc_last_body, grid=(GRID,),
    in_specs=[_acc_spec, _row_spec, _deg_spec, _b_spec],
    out_specs=_row_spec, out_shape=_out_sd)


@jax.jit
def kernel(t, x, edge_index, W1, b1, W2, b2, W3, b3):
    del t
    src = edge_index[0]
    dst = edge_index[1]
    pad = EPAD - E
    srcp = jnp.concatenate(
        [src, jnp.zeros((pad,), jnp.int32)]).reshape(NW, NCHUNK, CH)
    dstp = jnp.concatenate(
        [dst, jnp.full((pad,), TRASH, jnp.int32)]).reshape(NW, NCHUNK, CH)

    xp = jnp.pad(x, ((0, NROWS - N), (0, 0)))
    zero_f = jnp.zeros((RPT, F), jnp.float32)
    ones_d = jnp.ones((CH, F), jnp.float32)
    b1r = b1.reshape(1, F)
    b2r = b2.reshape(1, F)
    b3r = b3.reshape(1, F)

    sc_deg = _sc_deg_kernel()
    sc_agg = _sc_agg_kernel()
    degp = sc_deg(dstp, zero_f, ones_d)

    hp1 = _tc_first(xp, W1, degp)
    acc1 = sc_agg(hp1, srcp, dstp, zero_f)
    hp2 = _tc_mid(acc1, hp1, degp, b1r, W2)
    acc2 = sc_agg(hp2, srcp, dstp, zero_f)
    hp3 = _tc_mid(acc2, hp2, degp, b2r, W3)
    acc3 = sc_agg(hp3, srcp, dstp, zero_f)
    out = _tc_last(acc3, hp3, degp, b3r)
    return out[:N]
